# baseline (speedup 1.0000x reference)
import functools
import jax
import jax.numpy as jnp
from jax import lax
from jax.experimental import pallas as pl
from jax.experimental.pallas import tpu as pltpu
from jax.experimental.pallas import tpu_sc as plsc

N = 16384
B = 16
TILE = 4096
GRID = N // TILE
Z128 = 128
LANES = 16
NSUB = 16
CHUNK = N // NSUB


def _tc_body(z_ref, pos_ref, emb_ref, Wp_ref, W1_ref, b1_ref, W2_ref,
             b2_ref, xa_ref, A_sc, P_sc):
    i = pl.program_id(0)

    @pl.when(i == 0)
    def _init():
        A_sc[...] = jnp.dot(emb_ref[...], W1_ref[...],
                            preferred_element_type=jnp.float32)
        P_sc[...] = jnp.dot(Wp_ref[...], W1_ref[...],
                            preferred_element_type=jnp.float32)

    zc = z_ref[0]
    lane = lax.broadcasted_iota(jnp.int32, (TILE, Z128), 1)
    oh_z = (zc == lane).astype(jnp.float32)
    a = jnp.dot(oh_z, A_sc[...], preferred_element_type=jnp.float32)
    p = jnp.dot(pos_ref[...], P_sc[...], preferred_element_type=jnp.float32)
    hpre = a + p + b1_ref[...]
    h = hpre * jax.nn.sigmoid(hpre)
    xa = jnp.dot(h, W2_ref[...], preferred_element_type=jnp.float32)
    xa_ref[0] = jnp.reshape(xa, (1, TILE)) + b2_ref[...]


def _tc_energies(z, pos, emb, Wp, W1, b1, W2, b2):
    D = emb.shape[1]
    H = W1.shape[1]
    emb_p = jnp.pad(emb, ((0, Z128 - emb.shape[0]), (0, 0)))
    pos_p = jnp.pad(pos, ((0, 0), (0, 5)))
    Wp_p = jnp.pad(Wp, ((0, 5), (0, 0)))
    z_in = z.reshape(GRID, TILE, 1).astype(jnp.int32)
    b1r = b1.reshape(1, H)
    b2r = b2.reshape(1, 1)

    return pl.pallas_call(
        _tc_body,
        grid=(GRID,),
        in_specs=[
            pl.BlockSpec((1, TILE, 1), lambda i: (i, 0, 0)),
            pl.BlockSpec((TILE, 8), lambda i: (i, 0)),
            pl.BlockSpec((Z128, D), lambda i: (0, 0)),
            pl.BlockSpec((8, D), lambda i: (0, 0)),
            pl.BlockSpec((D, H), lambda i: (0, 0)),
            pl.BlockSpec((1, H), lambda i: (0, 0)),
            pl.BlockSpec((H, 1), lambda i: (0, 0)),
            pl.BlockSpec((1, 1), lambda i: (0, 0)),
        ],
        out_specs=pl.BlockSpec((1, 1, TILE), lambda i: (i, 0, 0)),
        out_shape=jax.ShapeDtypeStruct((GRID, 1, TILE), jnp.float32),
        scratch_shapes=[
            pltpu.VMEM((Z128, H), jnp.float32),
            pltpu.VMEM((8, H), jnp.float32),
        ],
    )(z_in, pos_p, emb_p, Wp_p, W1, b1r, W2, b2r)


def _sc_segsum(xa, ids):
    mesh = plsc.VectorSubcoreMesh(core_axis_name="c", subcore_axis_name="s",
                                  num_cores=2, num_subcores=NSUB)
    run = functools.partial(
        pl.kernel,
        out_type=jax.ShapeDtypeStruct((LANES,), jnp.float32),
        mesh=mesh,
        compiler_params=pltpu.CompilerParams(needs_layout_passes=False),
        scratch_types=[
            pltpu.VMEM((CHUNK,), jnp.float32),
            pltpu.VMEM((CHUNK,), jnp.int32),
            pltpu.VMEM((B, LANES), jnp.float32),
            pltpu.VMEM((NSUB, B, LANES), jnp.float32),
            pltpu.VMEM((LANES,), jnp.float32),
            pltpu.VMEM_SHARED((NSUB, B, LANES), jnp.float32),
        ],
    )(_sc_segsum_body)
    return run(xa, ids)


def _sc_segsum_body(xa_hbm, ids_hbm, out_hbm, xa_v, ids_v, part_v, gather_v,
                    res_v, shared):
    cid = lax.axis_index("c")
    sid = lax.axis_index("s")

    @pl.when(cid == 0)
    def _work():
        base = sid * CHUNK
        pltpu.sync_copy(xa_hbm.at[pl.ds(base, CHUNK)], xa_v)
        pltpu.sync_copy(ids_hbm.at[pl.ds(base, CHUNK)], ids_v)

        def body(i, accs):
            v = xa_v[pl.ds(i * LANES, LANES)]
            d = ids_v[pl.ds(i * LANES, LANES)]
            return tuple(accs[b] + jnp.where(d == b, v, 0.0)
                         for b in range(B))

        init = tuple(jnp.zeros((LANES,), jnp.float32) for _ in range(B))
        accs = lax.fori_loop(0, CHUNK // LANES, body, init)

        for b in range(B):
            part_v[b] = accs[b]
        pltpu.sync_copy(part_v, shared.at[sid])

    plsc.subcore_barrier()

    @pl.when((cid == 0) & (sid == 0))
    def _out():
        pltpu.sync_copy(shared, gather_v)
        for b in range(B):
            m = gather_v[0, b]
            for t in range(1, NSUB):
                m = m + gather_v[t, b]
            part_v[b] = m
        row = lax.iota(jnp.int32, LANES)
        total = jnp.zeros((LANES,), jnp.float32)
        for l in range(LANES):
            col = jnp.full((LANES,), l, jnp.int32)
            total = total + plsc.load_gather(part_v, [row, col])
        res_v[...] = total
        pltpu.sync_copy(res_v, out_hbm)


def kernel(z, pos, batch, emb, Wp, W1, b1, W2, b2):
    xa = _tc_energies(z, pos, emb, Wp, W1, b1, W2, b2)
    out = _sc_segsum(xa.reshape(N), batch.astype(jnp.int32))
    return out.reshape(B, 1)


# speedup vs baseline: 1.0671x; 1.0671x over previous
import functools
import jax
import jax.numpy as jnp
from jax import lax
from jax.experimental import pallas as pl
from jax.experimental.pallas import tpu as pltpu
from jax.experimental.pallas import tpu_sc as plsc

N = 16384
B = 16
TILE = 2048
GRID = N // TILE
Z128 = 128
LANES = 16
NSUB = 16
CHUNK = N // NSUB


def _tc_body(z_ref, pos_ref, emb_ref, Wp_ref, W1_ref, b1_ref, W2_ref,
             b2_ref, xa_ref, A_sc, P_sc):
    i = pl.program_id(0)

    @pl.when(i == 0)
    def _init():
        A_sc[...] = jnp.dot(emb_ref[...], W1_ref[...],
                            preferred_element_type=jnp.float32)
        P_sc[...] = jnp.dot(Wp_ref[...], W1_ref[...],
                            preferred_element_type=jnp.float32)

    zc = z_ref[0]
    lane = lax.broadcasted_iota(jnp.int32, (TILE, Z128), 1)
    oh_z = (zc == lane).astype(jnp.float32)
    a = jnp.dot(oh_z, A_sc[...], preferred_element_type=jnp.float32)
    p = jnp.dot(pos_ref[...], P_sc[...], preferred_element_type=jnp.float32)
    hpre = a + p + b1_ref[...]
    h = hpre * jax.nn.sigmoid(hpre)
    xa = jnp.dot(h, W2_ref[...], preferred_element_type=jnp.float32)
    xa_ref[...] = jnp.reshape(xa, (TILE // 128, 128)) + b2_ref[...]


def _tc_energies(z, pos, emb, Wp, W1, b1, W2, b2):
    D = emb.shape[1]
    H = W1.shape[1]
    emb_p = jnp.pad(emb, ((0, Z128 - emb.shape[0]), (0, 0)))
    pos_p = jnp.pad(pos, ((0, 0), (0, 5)))
    Wp_p = jnp.pad(Wp, ((0, 5), (0, 0)))
    z_in = z.reshape(GRID, TILE, 1).astype(jnp.int32)
    b1r = b1.reshape(1, H)
    b2r = b2.reshape(1, 1)

    return pl.pallas_call(
        _tc_body,
        grid=(GRID,),
        in_specs=[
            pl.BlockSpec((1, TILE, 1), lambda i: (i, 0, 0)),
            pl.BlockSpec((TILE, 8), lambda i: (i, 0)),
            pl.BlockSpec((Z128, D), lambda i: (0, 0)),
            pl.BlockSpec((8, D), lambda i: (0, 0)),
            pl.BlockSpec((D, H), lambda i: (0, 0)),
            pl.BlockSpec((1, H), lambda i: (0, 0)),
            pl.BlockSpec((H, 1), lambda i: (0, 0)),
            pl.BlockSpec((1, 1), lambda i: (0, 0)),
        ],
        out_specs=pl.BlockSpec((TILE // 128, 128), lambda i: (i, 0)),
        out_shape=jax.ShapeDtypeStruct((N // 128, 128), jnp.float32),
        scratch_shapes=[
            pltpu.VMEM((Z128, H), jnp.float32),
            pltpu.VMEM((8, H), jnp.float32),
        ],
    )(z_in, pos_p, emb_p, Wp_p, W1, b1r, W2, b2r)


def _sc_segsum(xa, ids):
    mesh = plsc.VectorSubcoreMesh(core_axis_name="c", subcore_axis_name="s",
                                  num_cores=2, num_subcores=NSUB)
    run = functools.partial(
        pl.kernel,
        out_type=jax.ShapeDtypeStruct((LANES,), jnp.float32),
        mesh=mesh,
        compiler_params=pltpu.CompilerParams(needs_layout_passes=False),
        scratch_types=[
            pltpu.VMEM((CHUNK,), jnp.float32),
            pltpu.VMEM((CHUNK,), jnp.int32),
            pltpu.VMEM((B, LANES), jnp.float32),
            pltpu.VMEM((NSUB, B, LANES), jnp.float32),
            pltpu.VMEM((LANES,), jnp.float32),
            pltpu.VMEM_SHARED((NSUB, B, LANES), jnp.float32),
        ],
    )(_sc_segsum_body)
    return run(xa, ids)


def _sc_segsum_body(xa_hbm, ids_hbm, out_hbm, xa_v, ids_v, part_v, gather_v,
                    res_v, shared):
    cid = lax.axis_index("c")
    sid = lax.axis_index("s")

    @pl.when(cid == 0)
    def _work():
        base = sid * CHUNK
        pltpu.sync_copy(xa_hbm.at[pl.ds(base, CHUNK)], xa_v)
        pltpu.sync_copy(ids_hbm.at[pl.ds(base, CHUNK)], ids_v)

        def body(i, accs):
            v = xa_v[pl.ds(i * LANES, LANES)]
            d = ids_v[pl.ds(i * LANES, LANES)]
            return tuple(accs[b] + jnp.where(d == b, v, 0.0)
                         for b in range(B))

        init = tuple(jnp.zeros((LANES,), jnp.float32) for _ in range(B))
        accs = lax.fori_loop(0, CHUNK // LANES, body, init)

        for b in range(B):
            part_v[b] = accs[b]
        pltpu.sync_copy(part_v, shared.at[sid])

    plsc.subcore_barrier()

    @pl.when((cid == 0) & (sid == 0))
    def _out():
        pltpu.sync_copy(shared, gather_v)
        for b in range(B):
            m = gather_v[0, b]
            for t in range(1, NSUB):
                m = m + gather_v[t, b]
            part_v[b] = m
        row = lax.iota(jnp.int32, LANES)
        total = jnp.zeros((LANES,), jnp.float32)
        for l in range(LANES):
            col = jnp.full((LANES,), l, jnp.int32)
            total = total + plsc.load_gather(part_v, [row, col])
        res_v[...] = total
        pltpu.sync_copy(res_v, out_hbm)


def kernel(z, pos, batch, emb, Wp, W1, b1, W2, b2):
    xa = _tc_energies(z, pos, emb, Wp, W1, b1, W2, b2)
    out = _sc_segsum(xa.reshape(N), batch.astype(jnp.int32))
    return out.reshape(B, 1)


# speedup vs baseline: 1.0682x; 1.0010x over previous
import functools
import jax
import jax.numpy as jnp
from jax import lax
from jax.experimental import pallas as pl
from jax.experimental.pallas import tpu as pltpu
from jax.experimental.pallas import tpu_sc as plsc

N = 16384
B = 16
TILE = 2048
GRID = N // TILE
Z128 = 128
LANES = 16
NSUB = 16
CHUNK = N // NSUB


def _tc_body(z_ref, pos_ref, emb_ref, Wp_ref, W1_ref, b1_ref, W2_ref,
             b2_ref, xa_ref, A_sc, P_sc):
    i = pl.program_id(0)

    @pl.when(i == 0)
    def _init():
        A_sc[...] = jnp.dot(emb_ref[...], W1_ref[...],
                            preferred_element_type=jnp.float32)
        P_sc[...] = jnp.dot(Wp_ref[...], W1_ref[...],
                            preferred_element_type=jnp.float32)

    zc = z_ref[0]
    lane = lax.broadcasted_iota(jnp.int32, (TILE, Z128), 1)
    oh_z = (zc == lane).astype(jnp.float32)
    a = jnp.dot(oh_z, A_sc[...], preferred_element_type=jnp.float32)
    p = jnp.dot(pos_ref[...], P_sc[...], preferred_element_type=jnp.float32)
    hpre = a + p + b1_ref[...]
    h = hpre * jax.nn.sigmoid(hpre)
    xa = jnp.dot(h, W2_ref[...], preferred_element_type=jnp.float32)
    xa_row = jnp.reshape(xa, (1, TILE)) + b2_ref[...]
    for r in range(TILE // 128):
        xa_ref[r, :] = xa_row[0, r * 128:(r + 1) * 128]


def _tc_energies(z, pos, emb, Wp, W1, b1, W2, b2):
    D = emb.shape[1]
    H = W1.shape[1]
    emb_p = jnp.pad(emb, ((0, Z128 - emb.shape[0]), (0, 0)))
    pos_p = jnp.pad(pos, ((0, 0), (0, 5)))
    Wp_p = jnp.pad(Wp, ((0, 5), (0, 0)))
    z_in = z.reshape(GRID, TILE, 1).astype(jnp.int32)
    b1r = b1.reshape(1, H)
    b2r = b2.reshape(1, 1)

    return pl.pallas_call(
        _tc_body,
        grid=(GRID,),
        in_specs=[
            pl.BlockSpec((1, TILE, 1), lambda i: (i, 0, 0)),
            pl.BlockSpec((TILE, 8), lambda i: (i, 0)),
            pl.BlockSpec((Z128, D), lambda i: (0, 0)),
            pl.BlockSpec((8, D), lambda i: (0, 0)),
            pl.BlockSpec((D, H), lambda i: (0, 0)),
            pl.BlockSpec((1, H), lambda i: (0, 0)),
            pl.BlockSpec((H, 1), lambda i: (0, 0)),
            pl.BlockSpec((1, 1), lambda i: (0, 0)),
        ],
        out_specs=pl.BlockSpec((TILE // 128, 128), lambda i: (i, 0)),
        out_shape=jax.ShapeDtypeStruct((N // 128, 128), jnp.float32),
        scratch_shapes=[
            pltpu.VMEM((Z128, H), jnp.float32),
            pltpu.VMEM((8, H), jnp.float32),
        ],
    )(z_in, pos_p, emb_p, Wp_p, W1, b1r, W2, b2r)


def _sc_segsum(xa, ids):
    mesh = plsc.VectorSubcoreMesh(core_axis_name="c", subcore_axis_name="s",
                                  num_cores=2, num_subcores=NSUB)
    run = functools.partial(
        pl.kernel,
        out_type=jax.ShapeDtypeStruct((LANES,), jnp.float32),
        mesh=mesh,
        compiler_params=pltpu.CompilerParams(needs_layout_passes=False),
        scratch_types=[
            pltpu.VMEM((CHUNK,), jnp.float32),
            pltpu.VMEM((CHUNK,), jnp.int32),
            pltpu.VMEM((B, LANES), jnp.float32),
            pltpu.VMEM((NSUB, B, LANES), jnp.float32),
            pltpu.VMEM((LANES,), jnp.float32),
            pltpu.VMEM_SHARED((NSUB, B, LANES), jnp.float32),
        ],
    )(_sc_segsum_body)
    return run(xa, ids)


def _sc_segsum_body(xa_hbm, ids_hbm, out_hbm, xa_v, ids_v, part_v, gather_v,
                    res_v, shared):
    cid = lax.axis_index("c")
    sid = lax.axis_index("s")

    @pl.when(cid == 0)
    def _work():
        base = sid * CHUNK
        pltpu.sync_copy(xa_hbm.at[pl.ds(base, CHUNK)], xa_v)
        pltpu.sync_copy(ids_hbm.at[pl.ds(base, CHUNK)], ids_v)

        def body(i, accs):
            v = xa_v[pl.ds(i * LANES, LANES)]
            d = ids_v[pl.ds(i * LANES, LANES)]
            return tuple(accs[b] + jnp.where(d == b, v, 0.0)
                         for b in range(B))

        init = tuple(jnp.zeros((LANES,), jnp.float32) for _ in range(B))
        accs = lax.fori_loop(0, CHUNK // LANES, body, init)

        for b in range(B):
            part_v[b] = accs[b]
        pltpu.sync_copy(part_v, shared.at[sid])

    plsc.subcore_barrier()

    @pl.when((cid == 0) & (sid == 0))
    def _out():
        pltpu.sync_copy(shared, gather_v)
        for b in range(B):
            m = gather_v[0, b]
            for t in range(1, NSUB):
                m = m + gather_v[t, b]
            part_v[b] = m
        row = lax.iota(jnp.int32, LANES)
        total = jnp.zeros((LANES,), jnp.float32)
        for l in range(LANES):
            col = jnp.full((LANES,), l, jnp.int32)
            total = total + plsc.load_gather(part_v, [row, col])
        res_v[...] = total
        pltpu.sync_copy(res_v, out_hbm)


def kernel(z, pos, batch, emb, Wp, W1, b1, W2, b2):
    xa = _tc_energies(z, pos, emb, Wp, W1, b1, W2, b2)
    out = _sc_segsum(xa.reshape(N), batch.astype(jnp.int32))
    return out.reshape(B, 1)


# speedup vs baseline: 1.0963x; 1.0263x over previous
import functools
import jax
import jax.numpy as jnp
from jax import lax
from jax.experimental import pallas as pl
from jax.experimental.pallas import tpu as pltpu
from jax.experimental.pallas import tpu_sc as plsc

N = 16384
B = 16
TILE = 2048
GRID = N // TILE
Z128 = 128
LANES = 16
NSUB = 16
CHUNK = N // NSUB


def _tc_body(z_ref, pos_ref, emb_ref, Wp_ref, W1_ref, b1_ref, W2_ref,
             b2_ref, xa_ref, A_sc, P_sc):
    i = pl.program_id(0)

    @pl.when(i == 0)
    def _init():
        A_sc[...] = jnp.dot(emb_ref[...], W1_ref[...],
                            preferred_element_type=jnp.float32)
        P_sc[...] = jnp.dot(Wp_ref[...], W1_ref[...],
                            preferred_element_type=jnp.float32)

    zc = z_ref[0]
    lane = lax.broadcasted_iota(jnp.int32, (TILE, Z128), 1)
    oh_z = (zc == lane).astype(jnp.float32)
    a = jnp.dot(oh_z, A_sc[...], preferred_element_type=jnp.float32)
    p = jnp.dot(pos_ref[...], P_sc[...], preferred_element_type=jnp.float32)
    hpre = a + p + b1_ref[...]
    h = hpre * jax.nn.sigmoid(hpre)
    xa = jnp.dot(h, W2_ref[...], preferred_element_type=jnp.float32)
    xa_row = jnp.reshape(xa, (1, TILE)) + b2_ref[...]
    for r in range(TILE // 128):
        xa_ref[r, :] = xa_row[0, r * 128:(r + 1) * 128]


def _tc_energies(z, pos, emb, Wp, W1, b1, W2, b2):
    D = emb.shape[1]
    H = W1.shape[1]
    emb_p = jnp.pad(emb, ((0, Z128 - emb.shape[0]), (0, 0)))
    pos_p = jnp.pad(pos, ((0, 0), (0, 5)))
    Wp_p = jnp.pad(Wp, ((0, 5), (0, 0)))
    z_in = z.reshape(GRID, TILE, 1).astype(jnp.int32)
    b1r = b1.reshape(1, H)
    b2r = b2.reshape(1, 1)

    return pl.pallas_call(
        _tc_body,
        grid=(GRID,),
        in_specs=[
            pl.BlockSpec((1, TILE, 1), lambda i: (i, 0, 0)),
            pl.BlockSpec((TILE, 8), lambda i: (i, 0)),
            pl.BlockSpec((Z128, D), lambda i: (0, 0)),
            pl.BlockSpec((8, D), lambda i: (0, 0)),
            pl.BlockSpec((D, H), lambda i: (0, 0)),
            pl.BlockSpec((1, H), lambda i: (0, 0)),
            pl.BlockSpec((H, 1), lambda i: (0, 0)),
            pl.BlockSpec((1, 1), lambda i: (0, 0)),
        ],
        out_specs=pl.BlockSpec((TILE // 128, 128), lambda i: (i, 0)),
        out_shape=jax.ShapeDtypeStruct((N // 128, 128), jnp.float32),
        scratch_shapes=[
            pltpu.VMEM((Z128, H), jnp.float32),
            pltpu.VMEM((8, H), jnp.float32),
        ],
    )(z_in, pos_p, emb_p, Wp_p, W1, b1r, W2, b2r)


def _sc_segsum(xa, ids):
    mesh = plsc.VectorSubcoreMesh(core_axis_name="c", subcore_axis_name="s",
                                  num_cores=1, num_subcores=NSUB)
    run = functools.partial(
        pl.kernel,
        out_type=jax.ShapeDtypeStruct((LANES,), jnp.float32),
        mesh=mesh,
        compiler_params=pltpu.CompilerParams(needs_layout_passes=False),
        scratch_types=[
            pltpu.VMEM((CHUNK,), jnp.float32),
            pltpu.VMEM((CHUNK,), jnp.int32),
            pltpu.VMEM((B, LANES), jnp.float32),
            pltpu.VMEM((NSUB, B, LANES), jnp.float32),
            pltpu.VMEM((LANES,), jnp.float32),
            pltpu.VMEM_SHARED((NSUB, B, LANES), jnp.float32),
        ],
    )(_sc_segsum_body)
    return run(xa, ids)


def _sc_segsum_body(xa_hbm, ids_hbm, out_hbm, xa_v, ids_v, part_v, gather_v,
                    res_v, shared):
    cid = lax.axis_index("c")
    sid = lax.axis_index("s")

    @pl.when(cid == 0)
    def _work():
        base = sid * CHUNK
        pltpu.sync_copy(xa_hbm.at[pl.ds(base, CHUNK)], xa_v)
        pltpu.sync_copy(ids_hbm.at[pl.ds(base, CHUNK)], ids_v)

        def body(i, accs):
            v = xa_v[pl.ds(i * LANES, LANES)]
            d = ids_v[pl.ds(i * LANES, LANES)]
            return tuple(accs[b] + jnp.where(d == b, v, 0.0)
                         for b in range(B))

        init = tuple(jnp.zeros((LANES,), jnp.float32) for _ in range(B))
        accs = lax.fori_loop(0, CHUNK // LANES, body, init)

        for b in range(B):
            part_v[b] = accs[b]
        pltpu.sync_copy(part_v, shared.at[sid])

    plsc.subcore_barrier()

    @pl.when((cid == 0) & (sid == 0))
    def _out():
        pltpu.sync_copy(shared, gather_v)
        for b in range(B):
            m = gather_v[0, b]
            for t in range(1, NSUB):
                m = m + gather_v[t, b]
            part_v[b] = m
        row = lax.iota(jnp.int32, LANES)
        total = jnp.zeros((LANES,), jnp.float32)
        for l in range(LANES):
            col = jnp.full((LANES,), l, jnp.int32)
            total = total + plsc.load_gather(part_v, [row, col])
        res_v[...] = total
        pltpu.sync_copy(res_v, out_hbm)


def kernel(z, pos, batch, emb, Wp, W1, b1, W2, b2):
    xa = _tc_energies(z, pos, emb, Wp, W1, b1, W2, b2)
    out = _sc_segsum(xa.reshape(N), batch.astype(jnp.int32))
    return out.reshape(B, 1)


# speedup vs baseline: 1.7837x; 1.6270x over previous
import functools
import jax
import jax.numpy as jnp
from jax import lax
from jax.experimental import pallas as pl
from jax.experimental.pallas import tpu as pltpu
from jax.experimental.pallas import tpu_sc as plsc

N = 16384
B = 16
TILE = 2048
GRID = N // TILE
Z128 = 128
LANES = 16
NSUB = 16
CHUNK = N // NSUB


def _tc_body(z_ref, posT_ref, embT_ref, WpT_ref, W1T_ref, b1_ref, W2_ref,
             b2_ref, xa_ref, AT_sc, PT_sc):
    i = pl.program_id(0)

    @pl.when(i == 0)
    def _init():
        AT_sc[...] = jnp.dot(W1T_ref[...], embT_ref[...],
                             preferred_element_type=jnp.float32)
        PT_sc[...] = jnp.dot(W1T_ref[...], WpT_ref[...],
                             preferred_element_type=jnp.float32)

    z_row = z_ref[...]
    sub = lax.broadcasted_iota(jnp.int32, (Z128, TILE), 0)
    oh_zT = (sub == z_row).astype(jnp.float32)
    aT = jnp.dot(AT_sc[...], oh_zT, preferred_element_type=jnp.float32)
    pT = jnp.dot(PT_sc[...], posT_ref[...],
                 preferred_element_type=jnp.float32)
    hpreT = aT + pT + b1_ref[...]
    hT = hpreT * jax.nn.sigmoid(hpreT)
    xa_row = (jnp.dot(W2_ref[...], hT, preferred_element_type=jnp.float32)
              + b2_ref[...])
    for r in range(TILE // 128):
        xa_ref[r, :] = xa_row[0, r * 128:(r + 1) * 128]


def _tc_energies(z, pos, emb, Wp, W1, b1, W2, b2):
    D = emb.shape[1]
    H = W1.shape[1]
    embT_p = jnp.pad(emb.T, ((0, 0), (0, Z128 - emb.shape[0])))
    posT_p = jnp.pad(pos.T, ((0, 5), (0, 0)))
    WpT_p = jnp.pad(Wp.T, ((0, 0), (0, 5)))
    z_in = z.astype(jnp.int32).reshape(1, N)
    b1c = b1.reshape(H, 1)
    b2r = b2.reshape(1, 1)

    return pl.pallas_call(
        _tc_body,
        grid=(GRID,),
        in_specs=[
            pl.BlockSpec((1, TILE), lambda i: (0, i)),
            pl.BlockSpec((8, TILE), lambda i: (0, i)),
            pl.BlockSpec((D, Z128), lambda i: (0, 0)),
            pl.BlockSpec((D, 8), lambda i: (0, 0)),
            pl.BlockSpec((H, D), lambda i: (0, 0)),
            pl.BlockSpec((H, 1), lambda i: (0, 0)),
            pl.BlockSpec((1, H), lambda i: (0, 0)),
            pl.BlockSpec((1, 1), lambda i: (0, 0)),
        ],
        out_specs=pl.BlockSpec((TILE // 128, 128), lambda i: (i, 0)),
        out_shape=jax.ShapeDtypeStruct((N // 128, 128), jnp.float32),
        scratch_shapes=[
            pltpu.VMEM((H, Z128), jnp.float32),
            pltpu.VMEM((H, 8), jnp.float32),
        ],
    )(z_in, posT_p, embT_p, WpT_p, W1.T, b1c, W2.reshape(1, H), b2r)


def _sc_segsum(xa, ids):
    mesh = plsc.VectorSubcoreMesh(core_axis_name="c", subcore_axis_name="s",
                                  num_cores=1, num_subcores=NSUB)
    run = functools.partial(
        pl.kernel,
        out_type=jax.ShapeDtypeStruct((LANES,), jnp.float32),
        mesh=mesh,
        compiler_params=pltpu.CompilerParams(needs_layout_passes=False),
        scratch_types=[
            pltpu.VMEM((CHUNK,), jnp.float32),
            pltpu.VMEM((CHUNK,), jnp.int32),
            pltpu.VMEM((B, LANES), jnp.float32),
            pltpu.VMEM((NSUB, B, LANES), jnp.float32),
            pltpu.VMEM((LANES,), jnp.float32),
            pltpu.VMEM_SHARED((NSUB, B, LANES), jnp.float32),
        ],
    )(_sc_segsum_body)
    return run(xa, ids)


def _sc_segsum_body(xa_hbm, ids_hbm, out_hbm, xa_v, ids_v, part_v, gather_v,
                    res_v, shared):
    cid = lax.axis_index("c")
    sid = lax.axis_index("s")

    @pl.when(cid == 0)
    def _work():
        base = sid * CHUNK
        pltpu.sync_copy(xa_hbm.at[pl.ds(base, CHUNK)], xa_v)
        pltpu.sync_copy(ids_hbm.at[pl.ds(base, CHUNK)], ids_v)

        def body(i, accs):
            v = xa_v[pl.ds(i * LANES, LANES)]
            d = ids_v[pl.ds(i * LANES, LANES)]
            return tuple(accs[b] + jnp.where(d == b, v, 0.0)
                         for b in range(B))

        init = tuple(jnp.zeros((LANES,), jnp.float32) for _ in range(B))
        accs = lax.fori_loop(0, CHUNK // LANES, body, init)

        for b in range(B):
            part_v[b] = accs[b]
        pltpu.sync_copy(part_v, shared.at[sid])

    plsc.subcore_barrier()

    @pl.when((cid == 0) & (sid == 0))
    def _out():
        pltpu.sync_copy(shared, gather_v)
        for b in range(B):
            m = gather_v[0, b]
            for t in range(1, NSUB):
                m = m + gather_v[t, b]
            part_v[b] = m
        row = lax.iota(jnp.int32, LANES)
        total = jnp.zeros((LANES,), jnp.float32)
        for l in range(LANES):
            col = jnp.full((LANES,), l, jnp.int32)
            total = total + plsc.load_gather(part_v, [row, col])
        res_v[...] = total
        pltpu.sync_copy(res_v, out_hbm)


def kernel(z, pos, batch, emb, Wp, W1, b1, W2, b2):
    xa = _tc_energies(z, pos, emb, Wp, W1, b1, W2, b2)
    out = _sc_segsum(xa.reshape(N), batch.astype(jnp.int32))
    return out.reshape(B, 1)


# speedup vs baseline: 1.8699x; 1.0483x over previous
import functools
import jax
import jax.numpy as jnp
from jax import lax
from jax.experimental import pallas as pl
from jax.experimental.pallas import tpu as pltpu
from jax.experimental.pallas import tpu_sc as plsc

N = 16384
B = 16
TILE = 4096
GRID = N // TILE
Z128 = 128
LANES = 16
NSUB = 16
CHUNK = N // NSUB


def _tc_body(z_ref, posT_ref, embT_ref, WpT_ref, W1T_ref, b1_ref, W2_ref,
             b2_ref, xa_ref, AT_sc, PT_sc):
    i = pl.program_id(0)

    @pl.when(i == 0)
    def _init():
        AT_sc[...] = jnp.dot(W1T_ref[...], embT_ref[...],
                             preferred_element_type=jnp.float32)
        PT_sc[...] = jnp.dot(W1T_ref[...], WpT_ref[...],
                             preferred_element_type=jnp.float32)

    z_row = z_ref[...]
    sub = lax.broadcasted_iota(jnp.int32, (Z128, TILE), 0)
    oh_zT = (sub == z_row).astype(jnp.float32)
    aT = jnp.dot(AT_sc[...], oh_zT, preferred_element_type=jnp.float32)
    pT = jnp.dot(PT_sc[...], posT_ref[...],
                 preferred_element_type=jnp.float32)
    hpreT = aT + pT + b1_ref[...]
    hT = hpreT * jax.nn.sigmoid(hpreT)
    xa_row = (jnp.dot(W2_ref[...], hT, preferred_element_type=jnp.float32)
              + b2_ref[...])
    for r in range(TILE // 128):
        xa_ref[r, :] = xa_row[0, r * 128:(r + 1) * 128]


def _tc_energies(z, pos, emb, Wp, W1, b1, W2, b2):
    D = emb.shape[1]
    H = W1.shape[1]
    embT_p = jnp.pad(emb.T, ((0, 0), (0, Z128 - emb.shape[0])))
    posT_p = jnp.pad(pos.T, ((0, 5), (0, 0)))
    WpT_p = jnp.pad(Wp.T, ((0, 0), (0, 5)))
    z_in = z.astype(jnp.int32).reshape(1, N)
    b1c = b1.reshape(H, 1)
    b2r = b2.reshape(1, 1)

    return pl.pallas_call(
        _tc_body,
        grid=(GRID,),
        in_specs=[
            pl.BlockSpec((1, TILE), lambda i: (0, i)),
            pl.BlockSpec((8, TILE), lambda i: (0, i)),
            pl.BlockSpec((D, Z128), lambda i: (0, 0)),
            pl.BlockSpec((D, 8), lambda i: (0, 0)),
            pl.BlockSpec((H, D), lambda i: (0, 0)),
            pl.BlockSpec((H, 1), lambda i: (0, 0)),
            pl.BlockSpec((1, H), lambda i: (0, 0)),
            pl.BlockSpec((1, 1), lambda i: (0, 0)),
        ],
        out_specs=pl.BlockSpec((TILE // 128, 128), lambda i: (i, 0)),
        out_shape=jax.ShapeDtypeStruct((N // 128, 128), jnp.float32),
        scratch_shapes=[
            pltpu.VMEM((H, Z128), jnp.float32),
            pltpu.VMEM((H, 8), jnp.float32),
        ],
    )(z_in, posT_p, embT_p, WpT_p, W1.T, b1c, W2.reshape(1, H), b2r)


def _sc_segsum(xa, ids):
    mesh = plsc.VectorSubcoreMesh(core_axis_name="c", subcore_axis_name="s",
                                  num_cores=1, num_subcores=NSUB)
    run = functools.partial(
        pl.kernel,
        out_type=jax.ShapeDtypeStruct((LANES,), jnp.float32),
        mesh=mesh,
        compiler_params=pltpu.CompilerParams(needs_layout_passes=False),
        scratch_types=[
            pltpu.VMEM((CHUNK,), jnp.float32),
            pltpu.VMEM((CHUNK,), jnp.int32),
            pltpu.VMEM((B, LANES), jnp.float32),
            pltpu.VMEM((NSUB, B, LANES), jnp.float32),
            pltpu.VMEM((LANES,), jnp.float32),
            pltpu.VMEM_SHARED((NSUB, B, LANES), jnp.float32),
        ],
    )(_sc_segsum_body)
    return run(xa, ids)


def _sc_segsum_body(xa_hbm, ids_hbm, out_hbm, xa_v, ids_v, part_v, gather_v,
                    res_v, shared):
    cid = lax.axis_index("c")
    sid = lax.axis_index("s")

    @pl.when(cid == 0)
    def _work():
        base = sid * CHUNK
        pltpu.sync_copy(xa_hbm.at[pl.ds(base, CHUNK)], xa_v)
        pltpu.sync_copy(ids_hbm.at[pl.ds(base, CHUNK)], ids_v)

        def body(i, accs):
            v = xa_v[pl.ds(i * LANES, LANES)]
            d = ids_v[pl.ds(i * LANES, LANES)]
            return tuple(accs[b] + jnp.where(d == b, v, 0.0)
                         for b in range(B))

        init = tuple(jnp.zeros((LANES,), jnp.float32) for _ in range(B))
        accs = lax.fori_loop(0, CHUNK // LANES, body, init)

        for b in range(B):
            part_v[b] = accs[b]
        pltpu.sync_copy(part_v, shared.at[sid])

    plsc.subcore_barrier()

    @pl.when((cid == 0) & (sid == 0))
    def _out():
        pltpu.sync_copy(shared, gather_v)
        for b in range(B):
            m = gather_v[0, b]
            for t in range(1, NSUB):
                m = m + gather_v[t, b]
            part_v[b] = m
        row = lax.iota(jnp.int32, LANES)
        total = jnp.zeros((LANES,), jnp.float32)
        for l in range(LANES):
            col = jnp.full((LANES,), l, jnp.int32)
            total = total + plsc.load_gather(part_v, [row, col])
        res_v[...] = total
        pltpu.sync_copy(res_v, out_hbm)


def kernel(z, pos, batch, emb, Wp, W1, b1, W2, b2):
    xa = _tc_energies(z, pos, emb, Wp, W1, b1, W2, b2)
    out = _sc_segsum(xa.reshape(N), batch.astype(jnp.int32))
    return out.reshape(B, 1)
